# baseline (speedup 1.0000x reference)
import math
import jax
import jax.numpy as jnp
from jax.experimental import pallas as pl
from jax.experimental.pallas import tpu as pltpu

B = 32
LT = 64
LS = 256
N = LT + 2 * LS
C = 768
H = 12
DH = C // H
HID = 3072
KEEP = 180
NKEPT = LT + 2 * KEEP


def _layernorm(x, w, b, eps=1e-5):
    mu = jnp.mean(x, axis=-1, keepdims=True)
    var = jnp.mean((x - mu) ** 2, axis=-1, keepdims=True)
    return (x - mu) / jnp.sqrt(var + eps) * w + b


NQC = 4
QR = N // NQC


def _attn_proj_body(x_ref, qr_ref, qkv_ref, pw_ref, pb_ref,
                    st_ref, attn_ref, out_ref):
    qkv2d = qkv_ref[0]
    qrows = qr_ref[0]
    scale = DH ** -0.5
    xa_parts = []
    for h in range(H):
        qh = qrows[:, h * DH:(h + 1) * DH]
        kh = qkv2d[:, C + h * DH:C + (h + 1) * DH]
        vh = qkv2d[:, 2 * C + h * DH:2 * C + (h + 1) * DH]
        s0 = jax.lax.dot_general(qh, kh, (((1,), (1,)), ((), ())),
                                 preferred_element_type=jnp.float32)
        st_ref[0, h] = s0
        s = s0 * scale
        m = jnp.max(s, axis=1, keepdims=True)
        e = jnp.exp(s - m)
        denom = jnp.sum(e, axis=1, keepdims=True)
        p = e / denom
        attn_ref[0, h] = p
        xa_parts.append(jnp.dot(p, vh, preferred_element_type=jnp.float32))
    xa = jnp.concatenate(xa_parts, axis=1)
    part = jnp.dot(xa, pw_ref[...], preferred_element_type=jnp.float32)
    out_ref[0] = x_ref[0] + pb_ref[0] + part


def _attn_proj(x, qkv, proj_w, proj_b):
    pw = proj_w.T
    pb = proj_b.reshape(1, C)
    return pl.pallas_call(
        _attn_proj_body,
        grid=(B, NQC),
        in_specs=[
            pl.BlockSpec((1, QR, C), lambda b, c: (b, c, 0)),
            pl.BlockSpec((1, QR, 3 * C), lambda b, c: (b, c, 0)),
            pl.BlockSpec((1, N, 3 * C), lambda b, c: (b, 0, 0)),
            pl.BlockSpec((C, C), lambda b, c: (0, 0)),
            pl.BlockSpec((1, C), lambda b, c: (0, 0)),
        ],
        out_specs=[
            pl.BlockSpec((1, H, QR, N), lambda b, c: (b, 0, c, 0)),
            pl.BlockSpec((1, H, QR, N), lambda b, c: (b, 0, c, 0)),
            pl.BlockSpec((1, QR, C), lambda b, c: (b, c, 0)),
        ],
        out_shape=[
            jax.ShapeDtypeStruct((B, H, N, N), jnp.float32),
            jax.ShapeDtypeStruct((B, H, N, N), jnp.float32),
            jax.ShapeDtypeStruct((B, N, C), jnp.float32),
        ],
        compiler_params=pltpu.CompilerParams(
            vmem_limit_bytes=100 * 1024 * 1024),
    )(x, qkv, qkv, pw, pb)


def _gather_mlp_body(x2_ref, idx_ref, n2w_ref, n2b_ref, f1w_ref, f1b_ref,
                     f2w_ref, f2b_ref, out_ref):
    idx = idx_ref[0, 0, :NKEPT]
    onehot = (idx[:, None] ==
              jax.lax.broadcasted_iota(jnp.int32, (NKEPT, N), 1))
    g = jnp.dot(onehot.astype(jnp.float32), x2_ref[0],
                preferred_element_type=jnp.float32)
    mu = jnp.mean(g, axis=-1, keepdims=True)
    var = jnp.mean((g - mu) ** 2, axis=-1, keepdims=True)
    hn = (g - mu) / jnp.sqrt(var + 1e-5) * n2w_ref[0] + n2b_ref[0]
    a1 = jnp.dot(hn, f1w_ref[...],
                 preferred_element_type=jnp.float32) + f1b_ref[0]
    a1 = 0.5 * a1 * (1.0 + jax.lax.erf(a1 * (2.0 ** -0.5)))
    a2 = jnp.dot(a1, f2w_ref[...],
                 preferred_element_type=jnp.float32) + f2b_ref[0]
    out_ref[0] = g + a2


def _gather_mlp(x2, row_idx, n2w, n2b, f1w, f1b, f2w, f2b):
    idx_pad = jnp.pad(row_idx, ((0, 0), (0, 512 - NKEPT))).reshape(B, 1, 512)
    return pl.pallas_call(
        _gather_mlp_body,
        grid=(B,),
        in_specs=[
            pl.BlockSpec((1, N, C), lambda b: (b, 0, 0)),
            pl.BlockSpec((1, 1, 512), lambda b: (b, 0, 0)),
            pl.BlockSpec((1, C), lambda b: (0, 0)),
            pl.BlockSpec((1, C), lambda b: (0, 0)),
            pl.BlockSpec((C, HID), lambda b: (0, 0)),
            pl.BlockSpec((1, HID), lambda b: (0, 0)),
            pl.BlockSpec((HID, C), lambda b: (0, 0)),
            pl.BlockSpec((1, C), lambda b: (0, 0)),
        ],
        out_specs=pl.BlockSpec((1, NKEPT, C), lambda b: (b, 0, 0)),
        out_shape=jax.ShapeDtypeStruct((B, NKEPT, C), jnp.float32),
    )(x2, idx_pad, n2w.reshape(1, C), n2b.reshape(1, C),
      f1w.T, f1b.reshape(1, HID),
      f2w.T, f2b.reshape(1, C))


def kernel(x, global_index_template, global_index_ps, global_index_search,
           norm1_w, norm1_b, qkv_w, qkv_b, proj_w, proj_b,
           norm2_w, norm2_b, fc1_w, fc1_b, fc2_w, fc2_b):
    scale = DH ** -0.5

    h = _layernorm(x, norm1_w, norm1_b)
    qkv = h @ qkv_w.T + qkv_b

    s_raw, attn, x2 = _attn_proj(x, qkv, proj_w, proj_b)
    sc = s_raw * scale
    m_full = jax.lax.optimization_barrier(
        jnp.max(sc, axis=-1, keepdims=True))
    d_full = jax.lax.optimization_barrier(
        jnp.sum(jnp.exp(sc - m_full), axis=-1, keepdims=True))
    p_t = (jnp.exp(sc[:, :, :LT] - m_full[:, :, :LT])
           / d_full[:, :, :LT])
    p_t = jax.lax.optimization_barrier(p_t)
    attn_t = p_t[:, :, :, LT:].mean(axis=2).mean(axis=1)

    attn_t_ps = attn_t[:, :LS]
    attn_t_s = attn_t[:, LS:]
    idx_ps = jnp.argsort(-attn_t_ps, axis=1)
    idx_s = jnp.argsort(-attn_t_s, axis=1)
    topk_idx_ps = idx_ps[:, :KEEP]
    topk_idx_s = idx_s[:, :KEEP]
    keep_index_ps = jnp.take_along_axis(global_index_ps, topk_idx_ps, axis=1)
    removed_index_ps = jnp.take_along_axis(global_index_ps, idx_ps[:, KEEP:], axis=1)
    keep_index_s = jnp.take_along_axis(global_index_search, topk_idx_s, axis=1)
    removed_index_s = jnp.take_along_axis(global_index_search, idx_s[:, KEEP:], axis=1)

    row_idx = jnp.concatenate(
        [jnp.broadcast_to(jnp.arange(LT, dtype=jnp.int32), (B, LT)),
         topk_idx_ps + LT, topk_idx_s + LT + LS], axis=1)
    x_out = _gather_mlp(x2, row_idx, norm2_w, norm2_b,
                        fc1_w, fc1_b, fc2_w, fc2_b)

    return (x_out, global_index_template, keep_index_ps, keep_index_s,
            removed_index_ps, removed_index_s, attn)


# speedup vs baseline: 1.2476x; 1.2476x over previous
import math
import jax
import jax.numpy as jnp
from jax.experimental import pallas as pl
from jax.experimental.pallas import tpu as pltpu

B = 32
LT = 64
LS = 256
N = LT + 2 * LS
C = 768
H = 12
DH = C // H
HID = 3072
KEEP = 180
NKEPT = LT + 2 * KEEP


def _layernorm(x, w, b, eps=1e-5):
    mu = jnp.mean(x, axis=-1, keepdims=True)
    var = jnp.mean((x - mu) ** 2, axis=-1, keepdims=True)
    return (x - mu) / jnp.sqrt(var + eps) * w + b


NQC = 4
QR = N // NQC


def _attn_proj_body(x_ref, qr_ref, qkv_ref, pw_ref, pb_ref,
                    st_ref, attn_ref, out_ref):
    qkv2d = qkv_ref[0]
    qrows = qr_ref[0]
    scale = DH ** -0.5
    xa_parts = []
    for h in range(H):
        qh = qrows[:, h * DH:(h + 1) * DH]
        kh = qkv2d[:, C + h * DH:C + (h + 1) * DH]
        vh = qkv2d[:, 2 * C + h * DH:2 * C + (h + 1) * DH]
        s0 = jax.lax.dot_general(qh, kh, (((1,), (1,)), ((), ())),
                                 preferred_element_type=jnp.float32)

        s = s0 * scale
        m = jnp.max(s, axis=1, keepdims=True)
        e = jnp.exp(s - m)
        st_ref[0, h] = e
        denom = jnp.sum(e, axis=1, keepdims=True)
        p = e / denom
        attn_ref[0, h] = p
        xa_parts.append(jnp.dot(p, vh, preferred_element_type=jnp.float32))
    xa = jnp.concatenate(xa_parts, axis=1)
    part = jnp.dot(xa, pw_ref[...], preferred_element_type=jnp.float32)
    out_ref[0] = x_ref[0] + pb_ref[0] + part


def _attn_proj(x, qkv, proj_w, proj_b):
    pw = proj_w.T
    pb = proj_b.reshape(1, C)
    return pl.pallas_call(
        _attn_proj_body,
        grid=(B, NQC),
        in_specs=[
            pl.BlockSpec((1, QR, C), lambda b, c: (b, c, 0)),
            pl.BlockSpec((1, QR, 3 * C), lambda b, c: (b, c, 0)),
            pl.BlockSpec((1, N, 3 * C), lambda b, c: (b, 0, 0)),
            pl.BlockSpec((C, C), lambda b, c: (0, 0)),
            pl.BlockSpec((1, C), lambda b, c: (0, 0)),
        ],
        out_specs=[
            pl.BlockSpec((1, H, QR, N), lambda b, c: (b, 0, c, 0)),
            pl.BlockSpec((1, H, QR, N), lambda b, c: (b, 0, c, 0)),
            pl.BlockSpec((1, QR, C), lambda b, c: (b, c, 0)),
        ],
        out_shape=[
            jax.ShapeDtypeStruct((B, H, N, N), jnp.float32),
            jax.ShapeDtypeStruct((B, H, N, N), jnp.float32),
            jax.ShapeDtypeStruct((B, N, C), jnp.float32),
        ],
        compiler_params=pltpu.CompilerParams(
            vmem_limit_bytes=100 * 1024 * 1024),
    )(x, qkv, qkv, pw, pb)


def _gather_mlp_body(x2_ref, idx_ref, n2w_ref, n2b_ref, f1w_ref, f1b_ref,
                     f2w_ref, f2b_ref, out_ref):
    idx = idx_ref[0, 0, :NKEPT]
    onehot = (idx[:, None] ==
              jax.lax.broadcasted_iota(jnp.int32, (NKEPT, N), 1))
    g = jnp.dot(onehot.astype(jnp.float32), x2_ref[0],
                preferred_element_type=jnp.float32)
    mu = jnp.mean(g, axis=-1, keepdims=True)
    var = jnp.mean((g - mu) ** 2, axis=-1, keepdims=True)
    hn = (g - mu) / jnp.sqrt(var + 1e-5) * n2w_ref[0] + n2b_ref[0]
    a1 = jnp.dot(hn, f1w_ref[...],
                 preferred_element_type=jnp.float32) + f1b_ref[0]
    a1 = 0.5 * a1 * (1.0 + jax.lax.erf(a1 * (2.0 ** -0.5)))
    a2 = jnp.dot(a1, f2w_ref[...],
                 preferred_element_type=jnp.float32) + f2b_ref[0]
    out_ref[0] = g + a2


def _gather_mlp(x2, row_idx, n2w, n2b, f1w, f1b, f2w, f2b):
    idx_pad = jnp.pad(row_idx, ((0, 0), (0, 512 - NKEPT))).reshape(B, 1, 512)
    return pl.pallas_call(
        _gather_mlp_body,
        grid=(B,),
        in_specs=[
            pl.BlockSpec((1, N, C), lambda b: (b, 0, 0)),
            pl.BlockSpec((1, 1, 512), lambda b: (b, 0, 0)),
            pl.BlockSpec((1, C), lambda b: (0, 0)),
            pl.BlockSpec((1, C), lambda b: (0, 0)),
            pl.BlockSpec((C, HID), lambda b: (0, 0)),
            pl.BlockSpec((1, HID), lambda b: (0, 0)),
            pl.BlockSpec((HID, C), lambda b: (0, 0)),
            pl.BlockSpec((1, C), lambda b: (0, 0)),
        ],
        out_specs=pl.BlockSpec((1, NKEPT, C), lambda b: (b, 0, 0)),
        out_shape=jax.ShapeDtypeStruct((B, NKEPT, C), jnp.float32),
    )(x2, idx_pad, n2w.reshape(1, C), n2b.reshape(1, C),
      f1w.T, f1b.reshape(1, HID),
      f2w.T, f2b.reshape(1, C))


def kernel(x, global_index_template, global_index_ps, global_index_search,
           norm1_w, norm1_b, qkv_w, qkv_b, proj_w, proj_b,
           norm2_w, norm2_b, fc1_w, fc1_b, fc2_w, fc2_b):
    scale = DH ** -0.5

    h = _layernorm(x, norm1_w, norm1_b)
    qkv = h @ qkv_w.T + qkv_b

    e_full, attn, x2 = _attn_proj(x, qkv, proj_w, proj_b)
    d_full = jax.lax.optimization_barrier(
        jnp.sum(e_full, axis=-1, keepdims=True))
    p_t = e_full[:, :, :LT] / d_full[:, :, :LT]
    p_t = jax.lax.optimization_barrier(p_t)
    attn_t = p_t[:, :, :, LT:].mean(axis=2).mean(axis=1)

    attn_t_ps = attn_t[:, :LS]
    attn_t_s = attn_t[:, LS:]
    idx_ps = jnp.argsort(-attn_t_ps, axis=1)
    idx_s = jnp.argsort(-attn_t_s, axis=1)
    topk_idx_ps = idx_ps[:, :KEEP]
    topk_idx_s = idx_s[:, :KEEP]
    keep_index_ps = jnp.take_along_axis(global_index_ps, topk_idx_ps, axis=1)
    removed_index_ps = jnp.take_along_axis(global_index_ps, idx_ps[:, KEEP:], axis=1)
    keep_index_s = jnp.take_along_axis(global_index_search, topk_idx_s, axis=1)
    removed_index_s = jnp.take_along_axis(global_index_search, idx_s[:, KEEP:], axis=1)

    row_idx = jnp.concatenate(
        [jnp.broadcast_to(jnp.arange(LT, dtype=jnp.int32), (B, LT)),
         topk_idx_ps + LT, topk_idx_s + LT + LS], axis=1)
    x_out = _gather_mlp(x2, row_idx, norm2_w, norm2_b,
                        fc1_w, fc1_b, fc2_w, fc2_b)

    return (x_out, global_index_template, keep_index_ps, keep_index_s,
            removed_index_ps, removed_index_s, attn)


# speedup vs baseline: 1.5076x; 1.2084x over previous
import math
import jax
import jax.numpy as jnp
from jax.experimental import pallas as pl
from jax.experimental.pallas import tpu as pltpu

B = 32
LT = 64
LS = 256
N = LT + 2 * LS
C = 768
H = 12
DH = C // H
HID = 3072
KEEP = 180
NKEPT = LT + 2 * KEEP


def _layernorm(x, w, b, eps=1e-5):
    mu = jnp.mean(x, axis=-1, keepdims=True)
    var = jnp.mean((x - mu) ** 2, axis=-1, keepdims=True)
    return (x - mu) / jnp.sqrt(var + eps) * w + b


NQC = 2
QR = N // NQC


def _attn_proj_body(x_ref, qr_ref, qkv_ref, pw_ref, pb_ref,
                    st_ref, attn_ref, out_ref):
    qkv2d = qkv_ref[0]
    qrows = qr_ref[0]
    scale = DH ** -0.5
    xa_parts = []
    for h in range(H):
        qh = qrows[:, h * DH:(h + 1) * DH]
        kh = qkv2d[:, C + h * DH:C + (h + 1) * DH]
        vh = qkv2d[:, 2 * C + h * DH:2 * C + (h + 1) * DH]
        s0 = jax.lax.dot_general(qh, kh, (((1,), (1,)), ((), ())),
                                 preferred_element_type=jnp.float32)

        s = s0 * scale
        m = jnp.max(s, axis=1, keepdims=True)
        e = jnp.exp(s - m)
        st_ref[0, h] = e
        denom = jnp.sum(e, axis=1, keepdims=True)
        p = e / denom
        attn_ref[0, h] = p
        xa_parts.append(jnp.dot(p, vh, preferred_element_type=jnp.float32))
    xa = jnp.concatenate(xa_parts, axis=1)
    part = jnp.dot(xa, pw_ref[...], preferred_element_type=jnp.float32)
    out_ref[0] = x_ref[0] + pb_ref[0] + part


def _attn_proj(x, qkv, proj_w, proj_b):
    pw = proj_w.T
    pb = proj_b.reshape(1, C)
    return pl.pallas_call(
        _attn_proj_body,
        grid=(B, NQC),
        in_specs=[
            pl.BlockSpec((1, QR, C), lambda b, c: (b, c, 0)),
            pl.BlockSpec((1, QR, 3 * C), lambda b, c: (b, c, 0)),
            pl.BlockSpec((1, N, 3 * C), lambda b, c: (b, 0, 0)),
            pl.BlockSpec((C, C), lambda b, c: (0, 0)),
            pl.BlockSpec((1, C), lambda b, c: (0, 0)),
        ],
        out_specs=[
            pl.BlockSpec((1, H, QR, N), lambda b, c: (b, 0, c, 0)),
            pl.BlockSpec((1, H, QR, N), lambda b, c: (b, 0, c, 0)),
            pl.BlockSpec((1, QR, C), lambda b, c: (b, c, 0)),
        ],
        out_shape=[
            jax.ShapeDtypeStruct((B, H, N, N), jnp.float32),
            jax.ShapeDtypeStruct((B, H, N, N), jnp.float32),
            jax.ShapeDtypeStruct((B, N, C), jnp.float32),
        ],
        compiler_params=pltpu.CompilerParams(
            vmem_limit_bytes=100 * 1024 * 1024),
    )(x, qkv, qkv, pw, pb)


def _gather_mlp_body(x2_ref, idx_ref, n2w_ref, n2b_ref, f1w_ref, f1b_ref,
                     f2w_ref, f2b_ref, out_ref):
    idx = idx_ref[0, 0, :NKEPT]
    onehot = (idx[:, None] ==
              jax.lax.broadcasted_iota(jnp.int32, (NKEPT, N), 1))
    g = jnp.dot(onehot.astype(jnp.float32), x2_ref[0],
                preferred_element_type=jnp.float32)
    mu = jnp.mean(g, axis=-1, keepdims=True)
    var = jnp.mean((g - mu) ** 2, axis=-1, keepdims=True)
    hn = (g - mu) / jnp.sqrt(var + 1e-5) * n2w_ref[0] + n2b_ref[0]
    a1 = jnp.dot(hn, f1w_ref[...],
                 preferred_element_type=jnp.float32) + f1b_ref[0]
    a1 = 0.5 * a1 * (1.0 + jax.lax.erf(a1 * (2.0 ** -0.5)))
    a2 = jnp.dot(a1, f2w_ref[...],
                 preferred_element_type=jnp.float32) + f2b_ref[0]
    out_ref[0] = g + a2


def _gather_mlp(x2, row_idx, n2w, n2b, f1w, f1b, f2w, f2b):
    idx_pad = jnp.pad(row_idx, ((0, 0), (0, 512 - NKEPT))).reshape(B, 1, 512)
    return pl.pallas_call(
        _gather_mlp_body,
        grid=(B,),
        in_specs=[
            pl.BlockSpec((1, N, C), lambda b: (b, 0, 0)),
            pl.BlockSpec((1, 1, 512), lambda b: (b, 0, 0)),
            pl.BlockSpec((1, C), lambda b: (0, 0)),
            pl.BlockSpec((1, C), lambda b: (0, 0)),
            pl.BlockSpec((C, HID), lambda b: (0, 0)),
            pl.BlockSpec((1, HID), lambda b: (0, 0)),
            pl.BlockSpec((HID, C), lambda b: (0, 0)),
            pl.BlockSpec((1, C), lambda b: (0, 0)),
        ],
        out_specs=pl.BlockSpec((1, NKEPT, C), lambda b: (b, 0, 0)),
        out_shape=jax.ShapeDtypeStruct((B, NKEPT, C), jnp.float32),
    )(x2, idx_pad, n2w.reshape(1, C), n2b.reshape(1, C),
      f1w.T, f1b.reshape(1, HID),
      f2w.T, f2b.reshape(1, C))


def kernel(x, global_index_template, global_index_ps, global_index_search,
           norm1_w, norm1_b, qkv_w, qkv_b, proj_w, proj_b,
           norm2_w, norm2_b, fc1_w, fc1_b, fc2_w, fc2_b):
    scale = DH ** -0.5

    h = _layernorm(x, norm1_w, norm1_b)
    qkv = h @ qkv_w.T + qkv_b

    e_full, attn, x2 = _attn_proj(x, qkv, proj_w, proj_b)
    d_full = jax.lax.optimization_barrier(
        jnp.sum(e_full, axis=-1, keepdims=True))
    p_t = e_full[:, :, :LT] / d_full[:, :, :LT]
    p_t = jax.lax.optimization_barrier(p_t)
    attn_t = p_t[:, :, :, LT:].mean(axis=2).mean(axis=1)

    attn_t_ps = attn_t[:, :LS]
    attn_t_s = attn_t[:, LS:]
    idx_ps = jnp.argsort(-attn_t_ps, axis=1)
    idx_s = jnp.argsort(-attn_t_s, axis=1)
    topk_idx_ps = idx_ps[:, :KEEP]
    topk_idx_s = idx_s[:, :KEEP]
    keep_index_ps = jnp.take_along_axis(global_index_ps, topk_idx_ps, axis=1)
    removed_index_ps = jnp.take_along_axis(global_index_ps, idx_ps[:, KEEP:], axis=1)
    keep_index_s = jnp.take_along_axis(global_index_search, topk_idx_s, axis=1)
    removed_index_s = jnp.take_along_axis(global_index_search, idx_s[:, KEEP:], axis=1)

    row_idx = jnp.concatenate(
        [jnp.broadcast_to(jnp.arange(LT, dtype=jnp.int32), (B, LT)),
         topk_idx_ps + LT, topk_idx_s + LT + LS], axis=1)
    x_out = _gather_mlp(x2, row_idx, norm2_w, norm2_b,
                        fc1_w, fc1_b, fc2_w, fc2_b)

    return (x_out, global_index_template, keep_index_ps, keep_index_s,
            removed_index_ps, removed_index_s, attn)


# speedup vs baseline: 1.5448x; 1.0247x over previous
import math
import jax
import jax.numpy as jnp
from jax.experimental import pallas as pl
from jax.experimental.pallas import tpu as pltpu

B = 32
LT = 64
LS = 256
N = LT + 2 * LS
C = 768
H = 12
DH = C // H
HID = 3072
KEEP = 180
NKEPT = LT + 2 * KEEP


def _layernorm(x, w, b, eps=1e-5):
    mu = jnp.mean(x, axis=-1, keepdims=True)
    var = jnp.mean((x - mu) ** 2, axis=-1, keepdims=True)
    return (x - mu) / jnp.sqrt(var + eps) * w + b


NQC = 2
QR = N // NQC


def _attn_proj_body(x_ref, qkv_ref, pw_ref, pb_ref,
                    st_ref, attn_ref, out_ref):
    qkv2d = qkv_ref[0]
    c = pl.program_id(1)
    qrows = qkv_ref[0, pl.dslice(c * QR, QR), :]
    scale = DH ** -0.5
    xa_parts = []
    for h in range(H):
        qh = qrows[:, h * DH:(h + 1) * DH]
        kh = qkv2d[:, C + h * DH:C + (h + 1) * DH]
        vh = qkv2d[:, 2 * C + h * DH:2 * C + (h + 1) * DH]
        s0 = jax.lax.dot_general(qh, kh, (((1,), (1,)), ((), ())),
                                 preferred_element_type=jnp.float32)

        s = s0 * scale
        m = jnp.max(s, axis=1, keepdims=True)
        e = jnp.exp(s - m)
        st_ref[0, h] = e
        denom = jnp.sum(e, axis=1, keepdims=True)
        p = e / denom
        attn_ref[0, h] = p
        xa_parts.append(jnp.dot(p, vh, preferred_element_type=jnp.float32))
    xa = jnp.concatenate(xa_parts, axis=1)
    part = jnp.dot(xa, pw_ref[...], preferred_element_type=jnp.float32)
    out_ref[0] = x_ref[0] + pb_ref[0] + part


def _attn_proj(x, qkv, proj_w, proj_b):
    pw = proj_w.T
    pb = proj_b.reshape(1, C)
    return pl.pallas_call(
        _attn_proj_body,
        grid=(B, NQC),
        in_specs=[
            pl.BlockSpec((1, QR, C), lambda b, c: (b, c, 0)),
            pl.BlockSpec((1, N, 3 * C), lambda b, c: (b, 0, 0)),
            pl.BlockSpec((C, C), lambda b, c: (0, 0)),
            pl.BlockSpec((1, C), lambda b, c: (0, 0)),
        ],
        out_specs=[
            pl.BlockSpec((1, H, QR, N), lambda b, c: (b, 0, c, 0)),
            pl.BlockSpec((1, H, QR, N), lambda b, c: (b, 0, c, 0)),
            pl.BlockSpec((1, QR, C), lambda b, c: (b, c, 0)),
        ],
        out_shape=[
            jax.ShapeDtypeStruct((B, H, N, N), jnp.float32),
            jax.ShapeDtypeStruct((B, H, N, N), jnp.float32),
            jax.ShapeDtypeStruct((B, N, C), jnp.float32),
        ],
        compiler_params=pltpu.CompilerParams(
            vmem_limit_bytes=100 * 1024 * 1024),
    )(x, qkv, pw, pb)


def _gather_mlp_body(x2_ref, idx_ref, n2w_ref, n2b_ref, f1w_ref, f1b_ref,
                     f2w_ref, f2b_ref, out_ref):
    idx = idx_ref[0, 0, :NKEPT]
    onehot = (idx[:, None] ==
              jax.lax.broadcasted_iota(jnp.int32, (NKEPT, N), 1))
    g = jnp.dot(onehot.astype(jnp.float32), x2_ref[0],
                preferred_element_type=jnp.float32)
    mu = jnp.mean(g, axis=-1, keepdims=True)
    var = jnp.mean((g - mu) ** 2, axis=-1, keepdims=True)
    hn = (g - mu) / jnp.sqrt(var + 1e-5) * n2w_ref[0] + n2b_ref[0]
    a1 = jnp.dot(hn, f1w_ref[...],
                 preferred_element_type=jnp.float32) + f1b_ref[0]
    a1 = 0.5 * a1 * (1.0 + jax.lax.erf(a1 * (2.0 ** -0.5)))
    a2 = jnp.dot(a1, f2w_ref[...],
                 preferred_element_type=jnp.float32) + f2b_ref[0]
    out_ref[0] = g + a2


def _gather_mlp(x2, row_idx, n2w, n2b, f1w, f1b, f2w, f2b):
    idx_pad = jnp.pad(row_idx, ((0, 0), (0, 512 - NKEPT))).reshape(B, 1, 512)
    return pl.pallas_call(
        _gather_mlp_body,
        grid=(B,),
        in_specs=[
            pl.BlockSpec((1, N, C), lambda b: (b, 0, 0)),
            pl.BlockSpec((1, 1, 512), lambda b: (b, 0, 0)),
            pl.BlockSpec((1, C), lambda b: (0, 0)),
            pl.BlockSpec((1, C), lambda b: (0, 0)),
            pl.BlockSpec((C, HID), lambda b: (0, 0)),
            pl.BlockSpec((1, HID), lambda b: (0, 0)),
            pl.BlockSpec((HID, C), lambda b: (0, 0)),
            pl.BlockSpec((1, C), lambda b: (0, 0)),
        ],
        out_specs=pl.BlockSpec((1, NKEPT, C), lambda b: (b, 0, 0)),
        out_shape=jax.ShapeDtypeStruct((B, NKEPT, C), jnp.float32),
    )(x2, idx_pad, n2w.reshape(1, C), n2b.reshape(1, C),
      f1w.T, f1b.reshape(1, HID),
      f2w.T, f2b.reshape(1, C))


def kernel(x, global_index_template, global_index_ps, global_index_search,
           norm1_w, norm1_b, qkv_w, qkv_b, proj_w, proj_b,
           norm2_w, norm2_b, fc1_w, fc1_b, fc2_w, fc2_b):
    scale = DH ** -0.5

    h = _layernorm(x, norm1_w, norm1_b)
    qkv = h @ qkv_w.T + qkv_b

    e_full, attn, x2 = _attn_proj(x, qkv, proj_w, proj_b)
    d_full = jax.lax.optimization_barrier(
        jnp.sum(e_full, axis=-1, keepdims=True))
    p_t = e_full[:, :, :LT] / d_full[:, :, :LT]
    p_t = jax.lax.optimization_barrier(p_t)
    attn_t = p_t[:, :, :, LT:].mean(axis=2).mean(axis=1)

    attn_t_ps = attn_t[:, :LS]
    attn_t_s = attn_t[:, LS:]
    idx_ps = jnp.argsort(-attn_t_ps, axis=1)
    idx_s = jnp.argsort(-attn_t_s, axis=1)
    topk_idx_ps = idx_ps[:, :KEEP]
    topk_idx_s = idx_s[:, :KEEP]
    keep_index_ps = jnp.take_along_axis(global_index_ps, topk_idx_ps, axis=1)
    removed_index_ps = jnp.take_along_axis(global_index_ps, idx_ps[:, KEEP:], axis=1)
    keep_index_s = jnp.take_along_axis(global_index_search, topk_idx_s, axis=1)
    removed_index_s = jnp.take_along_axis(global_index_search, idx_s[:, KEEP:], axis=1)

    row_idx = jnp.concatenate(
        [jnp.broadcast_to(jnp.arange(LT, dtype=jnp.int32), (B, LT)),
         topk_idx_ps + LT, topk_idx_s + LT + LS], axis=1)
    x_out = _gather_mlp(x2, row_idx, norm2_w, norm2_b,
                        fc1_w, fc1_b, fc2_w, fc2_b)

    return (x_out, global_index_template, keep_index_ps, keep_index_s,
            removed_index_ps, removed_index_s, attn)
